# baseline (speedup 1.0000x reference)
import functools

import jax
import jax.numpy as jnp
from jax import lax
from jax.experimental import pallas as pl
from jax.experimental.pallas import tpu as pltpu
from jax.experimental.pallas import tpu_sc as plsc

B = 16384
L = 200
D = 32
LANES = 16
WPR = 16

NC = 2
NS = 16
NW = NC * NS

CHUNK = 80
G = 8
CPG = G * L // CHUNK
ROWS_PER_W = B // NW
NGRP = ROWS_PER_W // G
NPAIR = NGRP // 2
CHUNK_ROWS = B * L // CHUNK

V = 1000000
BR = 2048
GRP = 8 * BR
NBLK = -(-V // GRP)
NSLOT = NBLK * GRP


def _linearize_body(*refs):
    ts, out_ref = refs[:8], refs[8]
    p = lax.broadcasted_iota(jnp.int32, (256, 128), 0)
    q = lax.broadcasted_iota(jnp.int32, (256, 128), 1)
    same = (p // 32) == (q // 16)
    e_lo = (same & ((p % 32) == (q % 16))).astype(jnp.bfloat16)
    e_hi = (same & ((p % 32) == (q % 16) + 16)).astype(jnp.bfloat16)
    x = jnp.concatenate([t[...] for t in ts], axis=0).astype(jnp.bfloat16)
    dims = (((0,), (0,)), ((), ()))
    lo = lax.dot_general(x, e_lo, dims, preferred_element_type=jnp.float32)
    hi = lax.dot_general(x, e_hi, dims, preferred_element_type=jnp.float32)
    lo16 = lax.bitcast_convert_type(lo.astype(jnp.bfloat16), jnp.uint16)
    hi16 = lax.bitcast_convert_type(hi.astype(jnp.bfloat16), jnp.uint16)
    w = lo16.astype(jnp.uint32) | (hi16.astype(jnp.uint32) << 16)
    out_ref[...] = lax.bitcast_convert_type(w, jnp.float32)


@functools.cache
def _build_table_linearize():
    last_blk = (V - 1) // BR
    specs = [
        pl.BlockSpec((32, BR), (lambda j, k=k: (0, jnp.minimum(8 * j + k, last_blk))))
        for k in range(8)
    ]
    return pl.pallas_call(
        _linearize_body,
        grid=(NBLK,),
        in_specs=specs,
        out_specs=pl.BlockSpec((BR, 128), lambda j: (j, 0)),
        out_shape=jax.ShapeDtypeStruct((NBLK * BR, 128), jnp.float32),
    )


@functools.cache
def _build_nbow_pool():
    mesh = plsc.VectorSubcoreMesh(core_axis_name="c", subcore_axis_name="s")
    return functools.partial(
        pl.kernel,
        mesh=mesh,
        out_type=jax.ShapeDtypeStruct((B, D), jnp.float32),
        scratch_types=[
            pltpu.VMEM((CPG, CHUNK), jnp.int32),
            pltpu.VMEM((CPG, CHUNK), jnp.int32),
            pltpu.VMEM((G * L, WPR), jnp.float32),
            pltpu.VMEM((G * L, WPR), jnp.float32),
            pltpu.VMEM((G, D), jnp.float32),
            pltpu.SemaphoreType.DMA,
            pltpu.SemaphoreType.DMA,
            pltpu.SemaphoreType.DMA,
        ],
        compiler_params=pltpu.CompilerParams(
            use_tc_tiling_on_sc=False, needs_layout_passes=False),
    )(_nbow_pool)


def _nbow_pool(idx_hbm, table_hbm, out_hbm,
               idx_v0, idx_v1, rows_v0, rows_v1, out_v, gsem0, gsem1, isem):
    wid = lax.axis_index("s") * NC + lax.axis_index("c")
    inv_l = jnp.float32(1.0 / L)
    cbase = wid * (NGRP * CPG)

    def load_idx(g, ibuf):
        return pltpu.async_copy(idx_hbm.at[pl.ds(cbase + g * CPG, CPG)], ibuf, isem)

    def wait_idx(ibuf):
        pltpu.make_async_copy(idx_hbm.at[pl.ds(cbase, CPG)], ibuf, isem).wait()

    def fire_gathers(ibuf, rbuf, sem):
        for c in range(CPG):
            pltpu.async_copy(
                table_hbm.at[ibuf.at[c]], rbuf.at[pl.ds(c * CHUNK, CHUNK)], sem)

    def drain_gathers(ibuf, rbuf, sem):
        for c in range(CPG):
            pltpu.make_async_copy(
                table_hbm.at[ibuf.at[c]], rbuf.at[pl.ds(c * CHUNK, CHUNK)], sem).wait()

    def accumulate(rbuf, g):
        for r in range(G):
            base = r * L

            def acc_body(t, carry, base=base):
                accs = list(carry)
                row0 = base + t * 8
                for j in range(8):
                    w = plsc.bitcast(rbuf[row0 + j, :], jnp.uint32)
                    a = plsc.bitcast(w << jnp.uint32(16), jnp.float32)
                    b = plsc.bitcast(w & jnp.uint32(0xFFFF0000), jnp.float32)
                    accs[2 * (j % 4)] += a
                    accs[2 * (j % 4) + 1] += b
                return tuple(accs)

            zero = jnp.zeros((LANES,), jnp.float32)
            accs = lax.fori_loop(0, L // 8, acc_body, (zero,) * 8)
            h0 = (accs[0] + accs[2]) + (accs[4] + accs[6])
            h1 = (accs[1] + accs[3]) + (accs[5] + accs[7])
            out_v[r, pl.ds(0, LANES)] = h0 * inv_l
            out_v[r, pl.ds(LANES, LANES)] = h1 * inv_l
        pltpu.sync_copy(out_v, out_hbm.at[pl.ds(wid * ROWS_PER_W + g * G, G)])

    load_idx(0, idx_v0).wait()
    fire_gathers(idx_v0, rows_v0, gsem0)
    load_idx(1, idx_v1)

    def pair_body(i, carry):
        g0 = 2 * i
        not_last = i < NPAIR - 1

        wait_idx(idx_v1)
        fire_gathers(idx_v1, rows_v1, gsem1)

        drain_gathers(idx_v0, rows_v0, gsem0)

        @pl.when(not_last)
        def _():
            load_idx(g0 + 2, idx_v0)

        accumulate(rows_v0, g0)

        @pl.when(not_last)
        def _():
            wait_idx(idx_v0)
            fire_gathers(idx_v0, rows_v0, gsem0)

        drain_gathers(idx_v1, rows_v1, gsem1)

        @pl.when(not_last)
        def _():
            load_idx(g0 + 3, idx_v1)

        accumulate(rows_v1, g0 + 1)
        return carry

    lax.fori_loop(0, NPAIR, pair_body, 0)


def kernel(text_or_code, embedding_table):
    tt = embedding_table.T
    lin = _build_table_linearize()(tt, tt, tt, tt, tt, tt, tt, tt)
    table_lin = lin.reshape(NSLOT, WPR)
    r = text_or_code
    gidx = (r & ~(GRP - 1)) | ((r & (BR - 1)) << 3) | ((r >> 11) & 7)
    idx = gidx.reshape(CHUNK_ROWS, CHUNK)
    return _build_nbow_pool()(idx, table_lin)


# speedup vs baseline: 1.0696x; 1.0696x over previous
import functools

import jax
import jax.numpy as jnp
from jax import lax
from jax.experimental import pallas as pl
from jax.experimental.pallas import tpu as pltpu
from jax.experimental.pallas import tpu_sc as plsc

B = 16384
L = 200
D = 32
LANES = 16
WPR = 16

NC = 2
NS = 16
NW = NC * NS

CHUNK = 80
G = 8
CPG = G * L // CHUNK
ROWS_PER_W = B // NW
NGRP = ROWS_PER_W // G
NPAIR = NGRP // 2
CHUNK_ROWS = B * L // CHUNK

V = 1000000
BR = 4096
SH = BR.bit_length() - 1
GRP = 8 * BR
NBLK = -(-V // GRP)
NSLOT = NBLK * GRP


def _linearize_body(*refs):
    ts, out_ref = refs[:8], refs[8]
    p = lax.broadcasted_iota(jnp.int32, (256, 128), 0)
    q = lax.broadcasted_iota(jnp.int32, (256, 128), 1)
    same = (p // 32) == (q // 16)
    e_lo = (same & ((p % 32) == (q % 16))).astype(jnp.bfloat16)
    e_hi = (same & ((p % 32) == (q % 16) + 16)).astype(jnp.bfloat16)
    x = jnp.concatenate([t[...] for t in ts], axis=0).astype(jnp.bfloat16)
    dims = (((0,), (0,)), ((), ()))
    lo = lax.dot_general(x, e_lo, dims, preferred_element_type=jnp.float32)
    hi = lax.dot_general(x, e_hi, dims, preferred_element_type=jnp.float32)
    lo16 = lax.bitcast_convert_type(lo.astype(jnp.bfloat16), jnp.uint16)
    hi16 = lax.bitcast_convert_type(hi.astype(jnp.bfloat16), jnp.uint16)
    w = lo16.astype(jnp.uint32) | (hi16.astype(jnp.uint32) << 16)
    out_ref[...] = lax.bitcast_convert_type(w, jnp.float32)


@functools.cache
def _build_table_linearize():
    last_blk = (V - 1) // BR
    specs = [
        pl.BlockSpec((32, BR), (lambda j, k=k: (0, jnp.minimum(8 * j + k, last_blk))))
        for k in range(8)
    ]
    return pl.pallas_call(
        _linearize_body,
        grid=(NBLK,),
        in_specs=specs,
        out_specs=pl.BlockSpec((BR, 128), lambda j: (j, 0)),
        out_shape=jax.ShapeDtypeStruct((NBLK * BR, 128), jnp.float32),
    )


@functools.cache
def _build_nbow_pool():
    mesh = plsc.VectorSubcoreMesh(core_axis_name="c", subcore_axis_name="s")
    return functools.partial(
        pl.kernel,
        mesh=mesh,
        out_type=jax.ShapeDtypeStruct((B, D), jnp.float32),
        scratch_types=[
            pltpu.VMEM((CPG, CHUNK), jnp.int32),
            pltpu.VMEM((CPG, CHUNK), jnp.int32),
            pltpu.VMEM((G * L, WPR), jnp.float32),
            pltpu.VMEM((G * L, WPR), jnp.float32),
            pltpu.VMEM((G, D), jnp.float32),
            pltpu.SemaphoreType.DMA,
            pltpu.SemaphoreType.DMA,
            pltpu.SemaphoreType.DMA,
        ],
        compiler_params=pltpu.CompilerParams(
            use_tc_tiling_on_sc=False, needs_layout_passes=False),
    )(_nbow_pool)


def _nbow_pool(idx_hbm, table_hbm, out_hbm,
               idx_v0, idx_v1, rows_v0, rows_v1, out_v, gsem0, gsem1, isem):
    wid = lax.axis_index("s") * NC + lax.axis_index("c")
    inv_l = jnp.float32(1.0 / L)
    cbase = wid * (NGRP * CPG)

    def load_idx(g, ibuf):
        return pltpu.async_copy(idx_hbm.at[pl.ds(cbase + g * CPG, CPG)], ibuf, isem)

    def wait_idx(ibuf):
        pltpu.make_async_copy(idx_hbm.at[pl.ds(cbase, CPG)], ibuf, isem).wait()

    def fire_gathers(ibuf, rbuf, sem):
        for c in range(CPG):
            pltpu.async_copy(
                table_hbm.at[ibuf.at[c]], rbuf.at[pl.ds(c * CHUNK, CHUNK)], sem)

    def drain_gathers(ibuf, rbuf, sem):
        for c in range(CPG):
            pltpu.make_async_copy(
                table_hbm.at[ibuf.at[c]], rbuf.at[pl.ds(c * CHUNK, CHUNK)], sem).wait()

    def accumulate(rbuf, g):
        for r in range(G):
            base = r * L

            def acc_body(t, carry, base=base):
                accs = list(carry)
                row0 = base + t * 8
                for j in range(8):
                    w = plsc.bitcast(rbuf[row0 + j, :], jnp.uint32)
                    a = plsc.bitcast(w << jnp.uint32(16), jnp.float32)
                    b = plsc.bitcast(w & jnp.uint32(0xFFFF0000), jnp.float32)
                    accs[2 * (j % 4)] += a
                    accs[2 * (j % 4) + 1] += b
                return tuple(accs)

            zero = jnp.zeros((LANES,), jnp.float32)
            accs = lax.fori_loop(0, L // 8, acc_body, (zero,) * 8)
            h0 = (accs[0] + accs[2]) + (accs[4] + accs[6])
            h1 = (accs[1] + accs[3]) + (accs[5] + accs[7])
            out_v[r, pl.ds(0, LANES)] = h0 * inv_l
            out_v[r, pl.ds(LANES, LANES)] = h1 * inv_l
        pltpu.sync_copy(out_v, out_hbm.at[pl.ds(wid * ROWS_PER_W + g * G, G)])

    load_idx(0, idx_v0).wait()
    fire_gathers(idx_v0, rows_v0, gsem0)
    load_idx(1, idx_v1)

    def pair_body(i, carry):
        g0 = 2 * i
        not_last = i < NPAIR - 1

        wait_idx(idx_v1)
        fire_gathers(idx_v1, rows_v1, gsem1)

        drain_gathers(idx_v0, rows_v0, gsem0)

        @pl.when(not_last)
        def _():
            load_idx(g0 + 2, idx_v0)

        accumulate(rows_v0, g0)

        @pl.when(not_last)
        def _():
            wait_idx(idx_v0)
            fire_gathers(idx_v0, rows_v0, gsem0)

        drain_gathers(idx_v1, rows_v1, gsem1)

        @pl.when(not_last)
        def _():
            load_idx(g0 + 3, idx_v1)

        accumulate(rows_v1, g0 + 1)
        return carry

    lax.fori_loop(0, NPAIR, pair_body, 0)


def kernel(text_or_code, embedding_table):
    tt = embedding_table.T
    lin = _build_table_linearize()(tt, tt, tt, tt, tt, tt, tt, tt)
    table_lin = lin.reshape(NSLOT, WPR)
    r = text_or_code.reshape(CHUNK_ROWS, CHUNK)
    idx = (r & ~(GRP - 1)) | ((r & (BR - 1)) << 3) | ((r >> SH) & 7)
    return _build_nbow_pool()(idx, table_lin)


# speedup vs baseline: 1.1019x; 1.0301x over previous
import functools

import jax
import jax.numpy as jnp
from jax import lax
from jax.experimental import pallas as pl
from jax.experimental.pallas import tpu as pltpu
from jax.experimental.pallas import tpu_sc as plsc

B = 16384
L = 200
D = 32
LANES = 16
WPR = 16

NC = 2
NS = 16
NW = NC * NS

CHUNK = 128
G = 16
CPG = G * L // CHUNK
ROWS_PER_W = B // NW
NGRP = ROWS_PER_W // G
NPAIR = NGRP // 2
CHUNK_ROWS = B * L // CHUNK

V = 1000000
BR = 4096
SH = BR.bit_length() - 1
GRP = 8 * BR
NBLK = -(-V // GRP)
NSLOT = NBLK * GRP


def _linearize_body(*refs):
    ts, out_ref = refs[:8], refs[8]
    p = lax.broadcasted_iota(jnp.int32, (256, 128), 0)
    q = lax.broadcasted_iota(jnp.int32, (256, 128), 1)
    same = (p // 32) == (q // 16)
    e_lo = (same & ((p % 32) == (q % 16))).astype(jnp.bfloat16)
    e_hi = (same & ((p % 32) == (q % 16) + 16)).astype(jnp.bfloat16)
    x = jnp.concatenate([t[...] for t in ts], axis=0).astype(jnp.bfloat16)
    dims = (((0,), (0,)), ((), ()))
    lo = lax.dot_general(x, e_lo, dims, preferred_element_type=jnp.float32)
    hi = lax.dot_general(x, e_hi, dims, preferred_element_type=jnp.float32)
    lo16 = lax.bitcast_convert_type(lo.astype(jnp.bfloat16), jnp.uint16)
    hi16 = lax.bitcast_convert_type(hi.astype(jnp.bfloat16), jnp.uint16)
    w = lo16.astype(jnp.uint32) | (hi16.astype(jnp.uint32) << 16)
    out_ref[...] = lax.bitcast_convert_type(w, jnp.float32)


@functools.cache
def _build_table_linearize():
    last_blk = (V - 1) // BR
    specs = [
        pl.BlockSpec((32, BR), (lambda j, k=k: (0, jnp.minimum(8 * j + k, last_blk))))
        for k in range(8)
    ]
    return pl.pallas_call(
        _linearize_body,
        grid=(NBLK,),
        in_specs=specs,
        out_specs=pl.BlockSpec((BR, 128), lambda j: (j, 0)),
        out_shape=jax.ShapeDtypeStruct((NBLK * BR, 128), jnp.float32),
    )


@functools.cache
def _build_nbow_pool():
    mesh = plsc.VectorSubcoreMesh(core_axis_name="c", subcore_axis_name="s")
    return functools.partial(
        pl.kernel,
        mesh=mesh,
        out_type=jax.ShapeDtypeStruct((B, D), jnp.float32),
        scratch_types=[
            pltpu.VMEM((CPG, CHUNK), jnp.int32),
            pltpu.VMEM((CPG, CHUNK), jnp.int32),
            pltpu.VMEM((G * L, WPR), jnp.float32),
            pltpu.VMEM((G * L, WPR), jnp.float32),
            pltpu.VMEM((G, D), jnp.float32),
            pltpu.SemaphoreType.DMA,
            pltpu.SemaphoreType.DMA,
            pltpu.SemaphoreType.DMA,
        ],
        compiler_params=pltpu.CompilerParams(
            use_tc_tiling_on_sc=False, needs_layout_passes=False),
    )(_nbow_pool)


def _nbow_pool(idx_hbm, table_hbm, out_hbm,
               idx_v0, idx_v1, rows_v0, rows_v1, out_v, gsem0, gsem1, isem):
    wid = lax.axis_index("s") * NC + lax.axis_index("c")
    inv_l = jnp.float32(1.0 / L)
    cbase = wid * (NGRP * CPG)

    def load_idx(g, ibuf):
        return pltpu.async_copy(idx_hbm.at[pl.ds(cbase + g * CPG, CPG)], ibuf, isem)

    def wait_idx(ibuf):
        pltpu.make_async_copy(idx_hbm.at[pl.ds(cbase, CPG)], ibuf, isem).wait()

    def fire_gathers(ibuf, rbuf, sem):
        for c in range(CPG):
            pltpu.async_copy(
                table_hbm.at[ibuf.at[c]], rbuf.at[pl.ds(c * CHUNK, CHUNK)], sem)

    def drain_gathers(ibuf, rbuf, sem):
        for c in range(CPG):
            pltpu.make_async_copy(
                table_hbm.at[ibuf.at[c]], rbuf.at[pl.ds(c * CHUNK, CHUNK)], sem).wait()

    def accumulate(rbuf, g):
        for r in range(G):
            base = r * L

            def acc_body(t, carry, base=base):
                accs = list(carry)
                row0 = base + t * 8
                for j in range(8):
                    w = plsc.bitcast(rbuf[row0 + j, :], jnp.uint32)
                    a = plsc.bitcast(w << jnp.uint32(16), jnp.float32)
                    b = plsc.bitcast(w & jnp.uint32(0xFFFF0000), jnp.float32)
                    accs[2 * (j % 4)] += a
                    accs[2 * (j % 4) + 1] += b
                return tuple(accs)

            zero = jnp.zeros((LANES,), jnp.float32)
            accs = lax.fori_loop(0, L // 8, acc_body, (zero,) * 8)
            h0 = (accs[0] + accs[2]) + (accs[4] + accs[6])
            h1 = (accs[1] + accs[3]) + (accs[5] + accs[7])
            out_v[r, pl.ds(0, LANES)] = h0 * inv_l
            out_v[r, pl.ds(LANES, LANES)] = h1 * inv_l
        pltpu.sync_copy(out_v, out_hbm.at[pl.ds(wid * ROWS_PER_W + g * G, G)])

    load_idx(0, idx_v0).wait()
    fire_gathers(idx_v0, rows_v0, gsem0)
    load_idx(1, idx_v1)

    def pair_body(i, carry):
        g0 = 2 * i
        not_last = i < NPAIR - 1

        wait_idx(idx_v1)
        fire_gathers(idx_v1, rows_v1, gsem1)

        drain_gathers(idx_v0, rows_v0, gsem0)

        @pl.when(not_last)
        def _():
            load_idx(g0 + 2, idx_v0)

        accumulate(rows_v0, g0)

        @pl.when(not_last)
        def _():
            wait_idx(idx_v0)
            fire_gathers(idx_v0, rows_v0, gsem0)

        drain_gathers(idx_v1, rows_v1, gsem1)

        @pl.when(not_last)
        def _():
            load_idx(g0 + 3, idx_v1)

        accumulate(rows_v1, g0 + 1)
        return carry

    lax.fori_loop(0, NPAIR, pair_body, 0)


def kernel(text_or_code, embedding_table):
    tt = embedding_table.T
    lin = _build_table_linearize()(tt, tt, tt, tt, tt, tt, tt, tt)
    table_lin = lin.reshape(NSLOT, WPR)
    r = text_or_code.reshape(CHUNK_ROWS, CHUNK)
    idx = (r & ~(GRP - 1)) | ((r & (BR - 1)) << 3) | ((r >> SH) & 7)
    return _build_nbow_pool()(idx, table_lin)
